# baseline (speedup 1.0000x reference)
import functools

import jax
import jax.numpy as jnp
from jax import lax
from jax.experimental import pallas as pl
from jax.experimental.pallas import tpu as pltpu
from jax.experimental.pallas import tpu_sc as plsc

EMB = 64
NCODES = 1024
NTOK = 64 * 576


ROWS = 3072


def _argmin_body(x_ref, w_ref, xsq_ref, wsq_ref, idx_ref):
    w2 = w_ref[...]
    x = x_ref[...]
    xsq = xsq_ref[...].reshape(1, ROWS)
    wsq = wsq_ref[...]
    dot_t = lax.dot_general(
        w2, x, dimension_numbers=(((0,), (1,)), ((), ())),
        preferred_element_type=jnp.float32)
    scores_t = (xsq + dot_t) + wsq
    kmin = jnp.min(scores_t, axis=0, keepdims=True)
    code = lax.broadcasted_iota(jnp.int32, scores_t.shape, 0).astype(
        jnp.float32)
    idxf = jnp.min(jnp.where(scores_t == kmin, code, float(NCODES)), axis=0)
    idx_ref[...] = idxf.astype(jnp.int32)


def _tc_argmin(x_flat, w2, xsq, wsq):
    ntok = x_flat.shape[0]
    grid = ntok // ROWS
    xsq = xsq.reshape(grid, 1, ROWS)
    return pl.pallas_call(
        _argmin_body,
        grid=(grid,),
        in_specs=[
            pl.BlockSpec((ROWS, EMB), lambda i: (i, 0)),
            pl.BlockSpec((EMB, NCODES), lambda i: (0, 0)),
            pl.BlockSpec((1, 1, ROWS), lambda i: (i, 0, 0)),
            pl.BlockSpec((NCODES, 1), lambda i: (0, 0)),
        ],
        out_specs=pl.BlockSpec((ROWS,), lambda i: (i,)),
        out_shape=jax.ShapeDtypeStruct((ntok,), jnp.int32),
    )(x_flat, w2, xsq, wsq)


_NC, _NS = 2, 16
_NW = _NC * _NS
_BPW = NTOK // _NW


_CHUNK = 576


def _sc_gather_body(table_hbm, idx_hbm, out_hbm, idx_v, rows_v, sem0,
                    sem1, sem2, sem3):
    wid = lax.axis_index("s") * _NC + lax.axis_index("c")
    base = wid * _BPW
    pltpu.sync_copy(idx_hbm.at[pl.ds(base, _CHUNK)], idx_v.at[0])
    g0 = pltpu.async_copy(table_hbm.at[idx_v.at[0]], rows_v.at[0], sem0)
    pltpu.sync_copy(idx_hbm.at[pl.ds(base + _CHUNK, _CHUNK)], idx_v.at[1])
    g1 = pltpu.async_copy(table_hbm.at[idx_v.at[1]], rows_v.at[1], sem1)
    g0.wait()
    w0 = pltpu.async_copy(rows_v.at[0], out_hbm.at[2 * wid], sem2)
    g1.wait()
    w1 = pltpu.async_copy(rows_v.at[1], out_hbm.at[2 * wid + 1], sem3)
    w0.wait()
    w1.wait()


@functools.cache
def _make_sc_gather():
    return functools.partial(
        pl.kernel,
        out_type=jax.ShapeDtypeStruct((64, 576, EMB), jnp.float32),
        mesh=plsc.VectorSubcoreMesh(
            core_axis_name="c", subcore_axis_name="s", num_cores=_NC,
            num_subcores=_NS),
        scratch_types=[
            pltpu.VMEM((2, _CHUNK), jnp.int32),
            pltpu.VMEM((2, _CHUNK, EMB), jnp.float32),
            pltpu.SemaphoreType.DMA,
            pltpu.SemaphoreType.DMA,
            pltpu.SemaphoreType.DMA,
            pltpu.SemaphoreType.DMA,
        ],
        compiler_params=pltpu.CompilerParams(use_tc_tiling_on_sc=False),
    )(_sc_gather_body)


def kernel(x, w):
    x_flat = x.reshape(NTOK, EMB)
    wsq = jnp.sum(w ** 2, axis=0).reshape(NCODES, 1)
    w2 = -2.0 * w
    table = w.T
    xsq = jnp.sum(x_flat ** 2, axis=1)
    idx = _tc_argmin(x_flat, w2, xsq, wsq)
    return _make_sc_gather()(table, idx)


# speedup vs baseline: 1.0716x; 1.0716x over previous
import functools

import jax
import jax.numpy as jnp
from jax import lax
from jax.experimental import pallas as pl
from jax.experimental.pallas import tpu as pltpu
from jax.experimental.pallas import tpu_sc as plsc

EMB = 64
NCODES = 1024
NTOK = 64 * 576


ROWS = 3072


def _argmin_body(x_ref, w_ref, xsq_ref, wsq_ref, idx_ref):
    w2 = w_ref[...]
    x = x_ref[...]
    xsq = xsq_ref[...].reshape(1, ROWS)
    wsq = wsq_ref[...]
    dot_t = lax.dot_general(
        w2, x, dimension_numbers=(((0,), (1,)), ((), ())),
        preferred_element_type=jnp.float32)
    scores_t = (xsq + dot_t) + wsq
    kmin = jnp.min(scores_t, axis=0, keepdims=True)
    code = lax.broadcasted_iota(jnp.int32, scores_t.shape, 0).astype(
        jnp.float32)
    idxf = jnp.min(jnp.where(scores_t == kmin, code, float(NCODES)), axis=0)
    idx_ref[...] = idxf.astype(jnp.int32)


def _tc_argmin(x_flat, w2, xsq, wsq):
    ntok = x_flat.shape[0]
    grid = ntok // ROWS
    xsq = xsq.reshape(grid, 1, ROWS)
    return pl.pallas_call(
        _argmin_body,
        grid=(grid,),
        in_specs=[
            pl.BlockSpec((ROWS, EMB), lambda i: (i, 0)),
            pl.BlockSpec((EMB, NCODES), lambda i: (0, 0)),
            pl.BlockSpec((1, 1, ROWS), lambda i: (i, 0, 0)),
            pl.BlockSpec((NCODES, 1), lambda i: (0, 0)),
        ],
        out_specs=pl.BlockSpec((ROWS,), lambda i: (i,)),
        out_shape=jax.ShapeDtypeStruct((ntok,), jnp.int32),
    )(x_flat, w2, xsq, wsq)


_NC, _NS = 2, 16
_NW = _NC * _NS
_BPW = NTOK // _NW


_CHUNK = 288


def _sc_gather_body(table_hbm, idx_hbm, out_hbm, idx_v, rows_v, sg0, sg1,
                    sw0, sw1):
    wid = lax.axis_index("s") * _NC + lax.axis_index("c")
    base = wid * _BPW
    pltpu.sync_copy(idx_hbm.at[pl.ds(base, _BPW)], idx_v)

    def gather(c, buf, sem):
        return pltpu.async_copy(
            table_hbm.at[idx_v.at[pl.ds(c * _CHUNK, _CHUNK)]],
            rows_v.at[buf], sem)

    def write(c, buf, sem):
        return pltpu.async_copy(
            rows_v.at[buf], out_hbm.at[pl.ds(base + c * _CHUNK, _CHUNK)],
            sem)

    g0 = gather(0, 0, sg0)
    g1 = gather(1, 1, sg1)
    g0.wait()
    w0 = write(0, 0, sw0)
    g1.wait()
    w1 = write(1, 1, sw1)
    w0.wait()
    g2 = gather(2, 0, sg0)
    w1.wait()
    g3 = gather(3, 1, sg1)
    g2.wait()
    w2 = write(2, 0, sw0)
    g3.wait()
    w3 = write(3, 1, sw1)
    w2.wait()
    w3.wait()


@functools.cache
def _make_sc_gather():
    return functools.partial(
        pl.kernel,
        out_type=jax.ShapeDtypeStruct((NTOK, 128), jnp.float32),
        mesh=plsc.VectorSubcoreMesh(
            core_axis_name="c", subcore_axis_name="s", num_cores=_NC,
            num_subcores=_NS),
        scratch_types=[
            pltpu.VMEM((_BPW,), jnp.int32),
            pltpu.VMEM((2, _CHUNK, 128), jnp.float32),
            pltpu.SemaphoreType.DMA,
            pltpu.SemaphoreType.DMA,
            pltpu.SemaphoreType.DMA,
            pltpu.SemaphoreType.DMA,
        ],
    )(_sc_gather_body)


def kernel(x, w):
    x_flat = x.reshape(NTOK, EMB)
    wsq = jnp.sum(w ** 2, axis=0).reshape(NCODES, 1)
    w2 = -2.0 * w
    table = jnp.zeros((NCODES, 128), jnp.float32).at[:, :EMB].set(w.T)
    xsq = jnp.sum(x_flat ** 2, axis=1)
    idx = _tc_argmin(x_flat, w2, xsq, wsq)
    out128 = _make_sc_gather()(table, idx)
    return out128[:, :EMB].reshape(64, 576, EMB)


# speedup vs baseline: 1.0848x; 1.0123x over previous
import functools

import jax
import jax.numpy as jnp
from jax import lax
from jax.experimental import pallas as pl
from jax.experimental.pallas import tpu as pltpu
from jax.experimental.pallas import tpu_sc as plsc

EMB = 64
NCODES = 1024
NTOK = 64 * 576


ROWS = 4096


def _argmin_body(x_ref, w_ref, xsq_ref, wsq_ref, idx_ref):
    w2 = w_ref[...]
    x = x_ref[...]
    xsq = xsq_ref[...].reshape(1, ROWS)
    wsq = wsq_ref[...]
    dot_t = lax.dot_general(
        w2, x, dimension_numbers=(((0,), (1,)), ((), ())),
        preferred_element_type=jnp.float32)
    scores_t = (xsq + dot_t) + wsq
    kmin = jnp.min(scores_t, axis=0, keepdims=True)
    code = lax.broadcasted_iota(jnp.int32, scores_t.shape, 0).astype(
        jnp.float32)
    idxf = jnp.min(jnp.where(scores_t == kmin, code, float(NCODES)), axis=0)
    idx_ref[...] = idxf.astype(jnp.int32)


def _tc_argmin(x_flat, w2, xsq, wsq):
    ntok = x_flat.shape[0]
    grid = ntok // ROWS
    xsq = xsq.reshape(grid, 1, ROWS)
    return pl.pallas_call(
        _argmin_body,
        grid=(grid,),
        in_specs=[
            pl.BlockSpec((ROWS, EMB), lambda i: (i, 0)),
            pl.BlockSpec((EMB, NCODES), lambda i: (0, 0)),
            pl.BlockSpec((1, 1, ROWS), lambda i: (i, 0, 0)),
            pl.BlockSpec((NCODES, 1), lambda i: (0, 0)),
        ],
        out_specs=pl.BlockSpec((ROWS,), lambda i: (i,)),
        out_shape=jax.ShapeDtypeStruct((ntok,), jnp.int32),
    )(x_flat, w2, xsq, wsq)


_NC, _NS = 2, 16
_NW = _NC * _NS
_BPW = NTOK // _NW


_CHUNK = 288


def _sc_gather_body(table_hbm, idx_hbm, out_hbm, idx_v, rows_v, sg0, sg1,
                    sw0, sw1):
    wid = lax.axis_index("s") * _NC + lax.axis_index("c")
    base = wid * _BPW
    pltpu.sync_copy(idx_hbm.at[pl.ds(base, _BPW)], idx_v)

    def gather(c, buf, sem):
        return pltpu.async_copy(
            table_hbm.at[idx_v.at[pl.ds(c * _CHUNK, _CHUNK)]],
            rows_v.at[buf], sem)

    def write(c, buf, sem):
        return pltpu.async_copy(
            rows_v.at[buf], out_hbm.at[pl.ds(base + c * _CHUNK, _CHUNK)],
            sem)

    g0 = gather(0, 0, sg0)
    g1 = gather(1, 1, sg1)
    g0.wait()
    w0 = write(0, 0, sw0)
    g1.wait()
    w1 = write(1, 1, sw1)
    w0.wait()
    g2 = gather(2, 0, sg0)
    w1.wait()
    g3 = gather(3, 1, sg1)
    g2.wait()
    w2 = write(2, 0, sw0)
    g3.wait()
    w3 = write(3, 1, sw1)
    w2.wait()
    w3.wait()


@functools.cache
def _make_sc_gather():
    return functools.partial(
        pl.kernel,
        out_type=jax.ShapeDtypeStruct((NTOK, 128), jnp.float32),
        mesh=plsc.VectorSubcoreMesh(
            core_axis_name="c", subcore_axis_name="s", num_cores=_NC,
            num_subcores=_NS),
        scratch_types=[
            pltpu.VMEM((_BPW,), jnp.int32),
            pltpu.VMEM((2, _CHUNK, 128), jnp.float32),
            pltpu.SemaphoreType.DMA,
            pltpu.SemaphoreType.DMA,
            pltpu.SemaphoreType.DMA,
            pltpu.SemaphoreType.DMA,
        ],
    )(_sc_gather_body)


def kernel(x, w):
    x_flat = x.reshape(NTOK, EMB)
    wsq = jnp.sum(w ** 2, axis=0).reshape(NCODES, 1)
    w2 = -2.0 * w
    table = lax.pad(w.T, jnp.float32(0), ((0, 0, 0), (0, 128 - EMB, 0)))
    xsq = jnp.sum(x_flat ** 2, axis=1)
    idx = _tc_argmin(x_flat, w2, xsq, wsq)
    out128 = _make_sc_gather()(table, idx)
    return out128[:, :EMB].reshape(64, 576, EMB)
